# baseline (speedup 1.0000x reference)
import functools

import jax
import jax.numpy as jnp
from jax import lax
from jax.experimental import pallas as pl
from jax.experimental.pallas import tpu as pltpu
from jax.experimental.pallas import tpu_sc as plsc

N_NODES = 10000
E_EDGES = 320000
D_IN = 128
D_OUT = 128

NC = 2
NS = 16
L = 16
NW = NC * NS
NPT = 320
N_PAD = NW * NPT
CH = 2000
GG = 64


def _make_segmax(C):
  mesh = plsc.VectorSubcoreMesh(core_axis_name="c", subcore_axis_name="s")

  @functools.partial(
      pl.kernel,
      out_type=jax.ShapeDtypeStruct((N_PAD, C), jnp.float32),
      mesh=mesh,
      scratch_types=[
          pltpu.VMEM((NPT, C), jnp.float32),
          pltpu.VMEM((CH,), jnp.int32),
          pltpu.VMEM((CH,), jnp.int32),
          pltpu.VMEM((CH + GG + L,), jnp.int32),
          pltpu.VMEM((CH + GG + L,), jnp.int32),
          pltpu.VMEM((GG, C), jnp.float32),
          pltpu.SemaphoreType.DMA,
      ],
      compiler_params=pltpu.CompilerParams(needs_layout_passes=False),
  )
  def segmax(b_hbm, src_hbm, dst_hbm, out_hbm,
             out_v, src_v, dst_v, sown_v, doff_v, rows_v, sem):
    wid = lax.axis_index("s") * NC + lax.axis_index("c")
    base = wid * NPT

    neg = jnp.full((L,), -jnp.inf, dtype=jnp.float32)

    def init_row(i, _):
      for cb in range(C // L):
        out_v[i, pl.ds(cb * L, L)] = neg
      return 0
    lax.fori_loop(0, NPT, init_row, 0)

    zero16 = jnp.zeros((L,), dtype=jnp.int32)

    def chunk_body(ci, _):
      pltpu.sync_copy(src_hbm.at[pl.ds(ci * CH, CH)], src_v)
      pltpu.sync_copy(dst_hbm.at[pl.ds(ci * CH, CH)], dst_v)

      def compact(g, cnt):
        d16 = dst_v[pl.ds(g * L, L)]
        s16 = src_v[pl.ds(g * L, L)]
        m = (d16 >= base) & (d16 < base + NPT)
        pos = plsc.cumsum(m.astype(jnp.int32))
        idx = jnp.where(m, cnt + pos - 1, CH + GG)
        plsc.store_scatter(sown_v, [idx], s16)
        plsc.store_scatter(doff_v, [idx], d16 - base)
        return cnt + pos[L - 1]

      cnt = lax.fori_loop(0, CH // L, compact, 0)

      def zpad(q, _):
        sown_v[pl.ds(cnt + q * L, L)] = zero16
        return 0
      lax.fori_loop(0, GG // L, zpad, 0)

      def group_body(gi, _):
        k0 = gi * GG
        pltpu.async_copy(b_hbm.at[sown_v.at[pl.ds(k0, GG)]], rows_v,
                         sem).wait()
        nk = jnp.minimum(GG, cnt - k0)

        def acc_edge(k, _):
          d = doff_v[pl.ds(k0 + k, L)][0]
          for cb in range(C // L):
            sl = pl.ds(cb * L, L)
            out_v[d, sl] = jnp.maximum(out_v[d, sl], rows_v[k, sl])
          return 0
        lax.fori_loop(0, nk, acc_edge, 0)
        return 0

      ngroups = (cnt + GG - 1) // GG
      lax.fori_loop(0, ngroups, group_body, 0)
      return 0

    lax.fori_loop(0, E_EDGES // CH, chunk_body, 0)

    pltpu.sync_copy(out_v, out_hbm.at[pl.ds(base, NPT)])

  return segmax


_segmax_gate = _make_segmax(2 * D_OUT)
_segmax_cand = _make_segmax(D_OUT)


_BM = 2000


def _k1_body(x_ref, h_ref, p_ref, ux_ref, uh_ref, up_ref, ba_ref,
             a_ref, b_ref):
  acc = jnp.dot(x_ref[...], ux_ref[...], preferred_element_type=jnp.float32)
  acc += jnp.dot(h_ref[...], uh_ref[...], preferred_element_type=jnp.float32)
  acc += jnp.dot(p_ref[...], up_ref[...], preferred_element_type=jnp.float32)
  half = acc.shape[1] // 2
  a_ref[...] = acc[:, :half] + ba_ref[...]
  b_ref[...] = acc[:, half:]


def _run_k1(x, h, posp, ux, uh, up, ba, cout):
  grid = N_NODES // _BM
  return pl.pallas_call(
      _k1_body,
      grid=(grid,),
      in_specs=[
          pl.BlockSpec((_BM, D_IN), lambda i: (i, 0)),
          pl.BlockSpec((_BM, D_OUT), lambda i: (i, 0)),
          pl.BlockSpec((_BM, 128), lambda i: (i, 0)),
          pl.BlockSpec((D_IN, 2 * cout), lambda i: (0, 0)),
          pl.BlockSpec((D_OUT, 2 * cout), lambda i: (0, 0)),
          pl.BlockSpec((128, 2 * cout), lambda i: (0, 0)),
          pl.BlockSpec((1, cout), lambda i: (0, 0)),
      ],
      out_specs=[
          pl.BlockSpec((_BM, cout), lambda i: (i, 0)),
          pl.BlockSpec((_BM, cout), lambda i: (i, 0)),
      ],
      out_shape=[
          jax.ShapeDtypeStruct((N_NODES, cout), jnp.float32),
          jax.ShapeDtypeStruct((N_NODES, cout), jnp.float32),
      ],
  )(x, h, posp, ux, uh, up, ba)


def _k2_body(x_ref, h_ref, p_ref, ag_ref, mg_ref, ux_ref, uh_ref, up_ref,
             ba_ref, a_ref, b_ref, u_ref):
  agg = ag_ref[...] + mg_ref[...]
  agg = jnp.where(jnp.isfinite(agg), agg, 0.0)
  gates = jax.nn.sigmoid(agg)
  r = gates[:, :D_OUT]
  u_ref[...] = gates[:, D_OUT:]
  hr = h_ref[...] * r
  acc = jnp.dot(x_ref[...], ux_ref[...], preferred_element_type=jnp.float32)
  acc += jnp.dot(hr, uh_ref[...], preferred_element_type=jnp.float32)
  acc += jnp.dot(p_ref[...], up_ref[...], preferred_element_type=jnp.float32)
  a_ref[...] = acc[:, :D_OUT] + ba_ref[...]
  b_ref[...] = acc[:, D_OUT:]


def _run_k2(x, h, posp, ag, mg, ux, uh, up, bc):
  grid = N_NODES // _BM
  return pl.pallas_call(
      _k2_body,
      grid=(grid,),
      in_specs=[
          pl.BlockSpec((_BM, D_IN), lambda i: (i, 0)),
          pl.BlockSpec((_BM, D_OUT), lambda i: (i, 0)),
          pl.BlockSpec((_BM, 128), lambda i: (i, 0)),
          pl.BlockSpec((_BM, 2 * D_OUT), lambda i: (i, 0)),
          pl.BlockSpec((_BM, 2 * D_OUT), lambda i: (i, 0)),
          pl.BlockSpec((D_IN, 2 * D_OUT), lambda i: (0, 0)),
          pl.BlockSpec((D_OUT, 2 * D_OUT), lambda i: (0, 0)),
          pl.BlockSpec((128, 2 * D_OUT), lambda i: (0, 0)),
          pl.BlockSpec((1, D_OUT), lambda i: (0, 0)),
      ],
      out_specs=[
          pl.BlockSpec((_BM, D_OUT), lambda i: (i, 0)),
          pl.BlockSpec((_BM, D_OUT), lambda i: (i, 0)),
          pl.BlockSpec((_BM, D_OUT), lambda i: (i, 0)),
      ],
      out_shape=[
          jax.ShapeDtypeStruct((N_NODES, D_OUT), jnp.float32),
          jax.ShapeDtypeStruct((N_NODES, D_OUT), jnp.float32),
          jax.ShapeDtypeStruct((N_NODES, D_OUT), jnp.float32),
      ],
  )(x, h, posp, ag, mg, ux, uh, up, bc)


def _k3_body(h_ref, ac_ref, mc_ref, u_ref, out_ref):
  agg = ac_ref[...] + mc_ref[...]
  agg = jnp.where(jnp.isfinite(agg), agg, 0.0)
  ht = jnp.tanh(agg)
  u = u_ref[...]
  out_ref[...] = (1.0 - u) * h_ref[...] + u * ht


def _run_k3(h, ac, mc, u):
  grid = N_NODES // _BM
  spec = pl.BlockSpec((_BM, D_OUT), lambda i: (i, 0))
  return pl.pallas_call(
      _k3_body,
      grid=(grid,),
      in_specs=[spec, spec, spec, spec],
      out_specs=spec,
      out_shape=jax.ShapeDtypeStruct((N_NODES, D_OUT), jnp.float32),
  )(h, ac, mc, u)


def _split_weights(W, b, cout):
  W1 = W[: D_IN + D_OUT]
  W2 = W[D_IN + D_OUT : 2 * (D_IN + D_OUT)]
  W3 = W[2 * (D_IN + D_OUT) :]
  Wd = W1 - W2
  ux = jnp.concatenate([Wd[:D_IN], W2[:D_IN]], axis=1)
  uh = jnp.concatenate([Wd[D_IN:], W2[D_IN:]], axis=1)
  w3p = jnp.pad(W3, ((0, 128 - 3), (0, 0)))
  up = jnp.concatenate([-w3p, w3p], axis=1)
  ba = b.reshape(1, -1)
  return ux, uh, up, ba


def kernel(h, x, pos, edge_index_gate, edge_index_cand, Wg, bg, Wc, bc):
  posp = jnp.pad(pos, ((0, 0), (0, 128 - pos.shape[1])))

  uxg, uhg, upg, bag = _split_weights(Wg, bg, 2 * D_OUT)
  uxc, uhc, upc, bac = _split_weights(Wc, bc, D_OUT)

  ag, bgt = _run_k1(x, h, posp, uxg, uhg, upg, bag, 2 * D_OUT)
  mg = _segmax_gate(bgt, edge_index_gate[0], edge_index_gate[1])[:N_NODES]

  ac, bct, u = _run_k2(x, h, posp, ag, mg, uxc, uhc, upc, bac)
  mc = _segmax_cand(bct, edge_index_cand[0], edge_index_cand[1])[:N_NODES]

  return _run_k3(h, ac, mc, u)
